# Initial kernel scaffold; baseline (speedup 1.0000x reference)
import jax
import jax.numpy as jnp
from jax.experimental import pallas as pl


def kernel(x_nchw, conv_w, gamma, beta):
    raise NotImplementedError("write your pallas kernel here")


# speedup vs baseline: 10.7111x; 10.7111x over previous
import numpy as np
import jax
import jax.numpy as jnp
from jax.experimental import pallas as pl
from jax.experimental.pallas import tpu as pltpu


def _interp_weights(n_in, n_out):
    a = np.zeros((n_out, n_in), dtype=np.float32)
    if n_in == 1:
        a[:, 0] = 1.0
    else:
        for i in range(n_out):
            src = i * (n_in - 1) / (n_out - 1)
            lo = min(int(np.floor(src)), n_in - 2)
            f = src - lo
            a[i, lo] += 1.0 - f
            a[i, lo + 1] += f
    return np.ascontiguousarray(a.T)


def _conv_stats_kernel(x_ref, w_ref, z_ref, s_ref, sq_ref):
    y = jnp.dot(w_ref[...], x_ref[0], preferred_element_type=jnp.float32)
    y = jnp.maximum(y, 0.0)
    z_ref[0] = y
    s_ref[0] = jnp.sum(y, axis=1, keepdims=True)
    sq_ref[0] = jnp.sum(y * y, axis=1, keepdims=True)


def _affine_upsample_kernel(z_ref, sc_ref, sh_ref, ahT_ref, awT_ref, o_ref):
    cout, h, w = z_ref.shape[1], z_ref.shape[2], z_ref.shape[3]
    h2 = ahT_ref.shape[1]
    w2 = awT_ref.shape[1]

    z = z_ref[0] * sc_ref[...] + sh_ref[...]
    zt = jnp.transpose(z, (0, 2, 1))
    q = jnp.dot(zt.reshape(cout * w, h), ahT_ref[...],
                preferred_element_type=jnp.float32)
    qt = jnp.transpose(q.reshape(cout, w, h2), (0, 2, 1))
    o = jnp.dot(qt.reshape(cout * h2, w), awT_ref[...],
                preferred_element_type=jnp.float32)
    o_ref[0] = o.reshape(cout, h2, w2)


def kernel(x_nchw, conv_w, gamma, beta, eps=1e-5):
    N, Cin, H, W = x_nchw.shape
    Cout = conv_w.shape[0]
    H2, W2 = 2 * H, 2 * W
    HW = H * W
    M = N * HW

    x3 = x_nchw.astype(jnp.float32).reshape(N, Cin, HW)
    wm = conv_w.reshape(Cout, Cin).astype(jnp.float32)

    z, s_part, sq_part = pl.pallas_call(
        _conv_stats_kernel,
        out_shape=(
            jax.ShapeDtypeStruct((N, Cout, HW), jnp.float32),
            jax.ShapeDtypeStruct((N, Cout, 1), jnp.float32),
            jax.ShapeDtypeStruct((N, Cout, 1), jnp.float32),
        ),
        grid=(N,),
        in_specs=[
            pl.BlockSpec((1, Cin, HW), lambda n: (n, 0, 0)),
            pl.BlockSpec((Cout, Cin), lambda n: (0, 0)),
        ],
        out_specs=[
            pl.BlockSpec((1, Cout, HW), lambda n: (n, 0, 0)),
            pl.BlockSpec((1, Cout, 1), lambda n: (n, 0, 0)),
            pl.BlockSpec((1, Cout, 1), lambda n: (n, 0, 0)),
        ],
        compiler_params=pltpu.CompilerParams(
            dimension_semantics=("parallel",)),
    )(x3, wm)

    s = jnp.sum(s_part, axis=0)[:, 0]
    sq = jnp.sum(sq_part, axis=0)[:, 0]
    mean = s / M
    var = jnp.maximum(sq / M - mean * mean, 0.0)
    scale = gamma.astype(jnp.float32) / jnp.sqrt(var + eps)
    shift = beta.astype(jnp.float32) - mean * scale

    ahT = jnp.asarray(_interp_weights(H, H2))
    awT = jnp.asarray(_interp_weights(W, W2))

    z4 = z.reshape(N, Cout, H, W)

    out = pl.pallas_call(
        _affine_upsample_kernel,
        out_shape=jax.ShapeDtypeStruct((N, Cout, H2, W2), jnp.float32),
        grid=(N,),
        in_specs=[
            pl.BlockSpec((1, Cout, H, W), lambda n: (n, 0, 0, 0)),
            pl.BlockSpec((Cout, 1, 1), lambda n: (0, 0, 0)),
            pl.BlockSpec((Cout, 1, 1), lambda n: (0, 0, 0)),
            pl.BlockSpec((H, H2), lambda n: (0, 0)),
            pl.BlockSpec((W, W2), lambda n: (0, 0)),
        ],
        out_specs=pl.BlockSpec((1, Cout, H2, W2), lambda n: (n, 0, 0, 0)),
        compiler_params=pltpu.CompilerParams(
            dimension_semantics=("parallel",)),
    )(z4, scale.reshape(Cout, 1, 1), shift.reshape(Cout, 1, 1), ahT, awT)

    return out


# speedup vs baseline: 31.4877x; 2.9397x over previous
import numpy as np
import jax
import jax.numpy as jnp
from jax.experimental import pallas as pl
from jax.experimental.pallas import tpu as pltpu


def _interp_taps(n_in, n_out):
    taps = []
    for i in range(n_out):
        if n_in == 1:
            taps.append((0, 1.0, 0.0))
            continue
        src = i * (n_in - 1) / (n_out - 1)
        lo = min(int(np.floor(src)), n_in - 2)
        f = src - lo
        taps.append((lo, 1.0 - f, f))
    return taps


def _interp_matrix_t(n_in, n_out):
    a = np.zeros((n_out, n_in), dtype=np.float32)
    for i, (lo, wl, wh) in enumerate(_interp_taps(n_in, n_out)):
        a[i, lo] += wl
        if wh:
            a[i, lo + 1] += wh
    return np.ascontiguousarray(a.T)


def _conv_stats_kernel(x_ref, w_ref, z_ref, s_ref, sq_ref):
    y = jnp.dot(x_ref[0], w_ref[...], preferred_element_type=jnp.float32)
    y = jnp.maximum(y, 0.0)
    z_ref[0] = y
    s_ref[0] = jnp.sum(y, axis=0, keepdims=True)
    sq_ref[0] = jnp.sum(y * y, axis=0, keepdims=True)


def _make_upsample_kernel(H, W, H2, W2, h_taps):
    def _kernel(z_ref, sc_ref, sh_ref, awT_ref, o_ref):
        c = z_ref.shape[2]
        t = z_ref[0] * sc_ref[...] + sh_ref[...]
        t3 = t.reshape(H, W, c)
        rows = []
        for lo, wl, wh in h_taps:
            r = t3[lo:lo + 1] * wl
            if wh:
                r = r + t3[lo + 1:lo + 2] * wh
            rows.append(r)
        v = jnp.concatenate(rows, axis=0)
        vt = jnp.transpose(v, (0, 2, 1))
        u = jnp.dot(vt.reshape(H2 * c, W), awT_ref[...],
                    preferred_element_type=jnp.float32)
        o_ref[0] = jnp.transpose(u.reshape(H2, c, W2), (0, 2, 1))
    return _kernel


def kernel(x_nchw, conv_w, gamma, beta, eps=1e-5):
    N, Cin, H, W = x_nchw.shape
    Cout = conv_w.shape[0]
    H2, W2 = 2 * H, 2 * W
    HW = H * W
    M = N * HW

    x_t = jnp.transpose(x_nchw.astype(jnp.float32), (0, 2, 3, 1))
    x3 = x_t.reshape(N, HW, Cin)
    wmT = conv_w.reshape(Cout, Cin).astype(jnp.float32).T

    z, s_part, sq_part = pl.pallas_call(
        _conv_stats_kernel,
        out_shape=(
            jax.ShapeDtypeStruct((N, HW, Cout), jnp.float32),
            jax.ShapeDtypeStruct((N, 1, Cout), jnp.float32),
            jax.ShapeDtypeStruct((N, 1, Cout), jnp.float32),
        ),
        grid=(N,),
        in_specs=[
            pl.BlockSpec((1, HW, Cin), lambda n: (n, 0, 0)),
            pl.BlockSpec((Cin, Cout), lambda n: (0, 0)),
        ],
        out_specs=[
            pl.BlockSpec((1, HW, Cout), lambda n: (n, 0, 0)),
            pl.BlockSpec((1, 1, Cout), lambda n: (n, 0, 0)),
            pl.BlockSpec((1, 1, Cout), lambda n: (n, 0, 0)),
        ],
        compiler_params=pltpu.CompilerParams(
            dimension_semantics=("parallel",)),
    )(x3, wmT)

    s = jnp.sum(s_part, axis=(0, 1))
    sq = jnp.sum(sq_part, axis=(0, 1))
    mean = s / M
    var = jnp.maximum(sq / M - mean * mean, 0.0)
    scale = gamma.astype(jnp.float32) / jnp.sqrt(var + eps)
    shift = beta.astype(jnp.float32) - mean * scale

    awT = jnp.asarray(_interp_matrix_t(W, W2))
    h_taps = _interp_taps(H, H2)

    out_t = pl.pallas_call(
        _make_upsample_kernel(H, W, H2, W2, h_taps),
        out_shape=jax.ShapeDtypeStruct((N, H2, W2, Cout), jnp.float32),
        grid=(N,),
        in_specs=[
            pl.BlockSpec((1, HW, Cout), lambda n: (n, 0, 0)),
            pl.BlockSpec((1, Cout), lambda n: (0, 0)),
            pl.BlockSpec((1, Cout), lambda n: (0, 0)),
            pl.BlockSpec((W, W2), lambda n: (0, 0)),
        ],
        out_specs=pl.BlockSpec((1, H2, W2, Cout), lambda n: (n, 0, 0, 0)),
        compiler_params=pltpu.CompilerParams(
            dimension_semantics=("parallel",)),
    )(z, scale.reshape(1, Cout), shift.reshape(1, Cout), awT)

    return jnp.transpose(out_t, (0, 3, 1, 2))


# speedup vs baseline: 43.3245x; 1.3759x over previous
import numpy as np
import jax
import jax.numpy as jnp
from jax.experimental import pallas as pl
from jax.experimental.pallas import tpu as pltpu


def _interp_taps(n_in, n_out):
    taps = []
    for i in range(n_out):
        if n_in == 1:
            taps.append((0, 1.0, 0.0))
            continue
        src = i * (n_in - 1) / (n_out - 1)
        lo = min(int(np.floor(src)), n_in - 2)
        f = src - lo
        taps.append((lo, 1.0 - f, f))
    return taps


def _interp_matrix_t(n_in, n_out):
    a = np.zeros((n_out, n_in), dtype=np.float32)
    for i, (lo, wl, wh) in enumerate(_interp_taps(n_in, n_out)):
        a[i, lo] += wl
        if wh:
            a[i, lo + 1] += wh
    return np.ascontiguousarray(a.T)


def _conv_stats_kernel(x_ref, w_ref, z_ref, s_ref, sq_ref):
    y = jnp.dot(x_ref[0], w_ref[...], preferred_element_type=jnp.float32)
    y = jnp.maximum(y, 0.0)
    z_ref[0] = y.astype(jnp.bfloat16)
    s_ref[0] = jnp.sum(y, axis=0, keepdims=True)
    sq_ref[0] = jnp.sum(y * y, axis=0, keepdims=True)


def _make_upsample_kernel(H, W, H2, W2, h_taps):
    def _kernel(z_ref, sc_ref, sh_ref, awT_ref, o_ref):
        c = z_ref.shape[2]
        t = z_ref[0] * sc_ref[...] + sh_ref[...]
        t3 = t.reshape(H, W, c)
        rows = []
        for lo, wl, wh in h_taps:
            r = t3[lo:lo + 1] * jnp.bfloat16(wl)
            if wh:
                r = r + t3[lo + 1:lo + 2] * jnp.bfloat16(wh)
            rows.append(r)
        v = jnp.concatenate(rows, axis=0)
        vt = jnp.transpose(v, (0, 2, 1))
        u = jnp.dot(vt.reshape(H2 * c, W), awT_ref[...],
                    preferred_element_type=jnp.float32)
        u = u.astype(jnp.bfloat16)
        o = jnp.transpose(u.reshape(H2, c, W2), (0, 2, 1))
        o_ref[0] = o.astype(jnp.float32)
    return _kernel


def kernel(x_nchw, conv_w, gamma, beta, eps=1e-5):
    N, Cin, H, W = x_nchw.shape
    Cout = conv_w.shape[0]
    H2, W2 = 2 * H, 2 * W
    HW = H * W
    M = N * HW

    x_t = jnp.transpose(x_nchw.astype(jnp.float32), (0, 2, 3, 1))
    x3 = x_t.reshape(N, HW, Cin)
    wmT = conv_w.reshape(Cout, Cin).astype(jnp.float32).T

    z, s_part, sq_part = pl.pallas_call(
        _conv_stats_kernel,
        out_shape=(
            jax.ShapeDtypeStruct((N, HW, Cout), jnp.bfloat16),
            jax.ShapeDtypeStruct((N, 1, Cout), jnp.float32),
            jax.ShapeDtypeStruct((N, 1, Cout), jnp.float32),
        ),
        grid=(N,),
        in_specs=[
            pl.BlockSpec((1, HW, Cin), lambda n: (n, 0, 0)),
            pl.BlockSpec((Cin, Cout), lambda n: (0, 0)),
        ],
        out_specs=[
            pl.BlockSpec((1, HW, Cout), lambda n: (n, 0, 0)),
            pl.BlockSpec((1, 1, Cout), lambda n: (n, 0, 0)),
            pl.BlockSpec((1, 1, Cout), lambda n: (n, 0, 0)),
        ],
        compiler_params=pltpu.CompilerParams(
            dimension_semantics=("parallel",)),
    )(x3, wmT)

    s = jnp.sum(s_part, axis=(0, 1))
    sq = jnp.sum(sq_part, axis=(0, 1))
    mean = s / M
    var = jnp.maximum(sq / M - mean * mean, 0.0)
    scale = gamma.astype(jnp.float32) / jnp.sqrt(var + eps)
    shift = beta.astype(jnp.float32) - mean * scale

    awT = jnp.asarray(_interp_matrix_t(W, W2)).astype(jnp.bfloat16)
    h_taps = _interp_taps(H, H2)

    out_t = pl.pallas_call(
        _make_upsample_kernel(H, W, H2, W2, h_taps),
        out_shape=jax.ShapeDtypeStruct((N, H2, W2, Cout), jnp.float32),
        grid=(N,),
        in_specs=[
            pl.BlockSpec((1, HW, Cout), lambda n: (n, 0, 0)),
            pl.BlockSpec((1, Cout), lambda n: (0, 0)),
            pl.BlockSpec((1, Cout), lambda n: (0, 0)),
            pl.BlockSpec((W, W2), lambda n: (0, 0)),
        ],
        out_specs=pl.BlockSpec((1, H2, W2, Cout), lambda n: (n, 0, 0, 0)),
        compiler_params=pltpu.CompilerParams(
            dimension_semantics=("parallel",)),
    )(z, scale.reshape(1, Cout).astype(jnp.bfloat16),
      shift.reshape(1, Cout).astype(jnp.bfloat16), awT)

    return jnp.transpose(out_t, (0, 3, 1, 2))


# speedup vs baseline: 53.6702x; 1.2388x over previous
import numpy as np
import jax
import jax.numpy as jnp
from jax.experimental import pallas as pl
from jax.experimental.pallas import tpu as pltpu


def _interp_taps(n_in, n_out):
    taps = []
    for i in range(n_out):
        if n_in == 1:
            taps.append((0, 1.0, 0.0))
            continue
        src = i * (n_in - 1) / (n_out - 1)
        lo = min(int(np.floor(src)), n_in - 2)
        f = src - lo
        taps.append((lo, 1.0 - f, f))
    return taps


def _interp_matrix_t(n_in, n_out):
    a = np.zeros((n_out, n_in), dtype=np.float32)
    for i, (lo, wl, wh) in enumerate(_interp_taps(n_in, n_out)):
        a[i, lo] += wl
        if wh:
            a[i, lo + 1] += wh
    return np.ascontiguousarray(a.T)


def _conv_stats_kernel(x_ref, w_ref, z_ref, s_ref, sq_ref):
    for i in range(x_ref.shape[0]):
        y = jax.lax.dot_general(
            x_ref[i], w_ref[...],
            dimension_numbers=(((1,), (1,)), ((), ())),
            preferred_element_type=jnp.float32)
        y = jnp.maximum(y, 0.0)
        z_ref[i] = y.astype(jnp.bfloat16)
        s_ref[i] = jnp.sum(y, axis=0, keepdims=True)
        sq_ref[i] = jnp.sum(y * y, axis=0, keepdims=True)


def _make_upsample_kernel(H, W, H2, W2, h_taps):
    def _kernel(z_ref, sc_ref, sh_ref, awT_ref, o_ref):
        c = z_ref.shape[2]
        for i in range(z_ref.shape[0]):
            t = z_ref[i] * sc_ref[...] + sh_ref[...]
            t3 = t.reshape(H, W, c)
            rows = []
            for lo, wl, wh in h_taps:
                r = t3[lo:lo + 1] * jnp.bfloat16(wl)
                if wh:
                    r = r + t3[lo + 1:lo + 2] * jnp.bfloat16(wh)
                rows.append(r)
            v = jnp.concatenate(rows, axis=0)
            vt = jnp.transpose(v, (0, 2, 1))
            u = jnp.dot(vt.reshape(H2 * c, W), awT_ref[...],
                        preferred_element_type=jnp.float32)
            u = u.astype(jnp.bfloat16)
            o = jnp.transpose(u.reshape(H2, c, W2), (0, 2, 1))
            o_ref[i] = o.astype(jnp.float32)
    return _kernel


def kernel(x_nchw, conv_w, gamma, beta, eps=1e-5):
    N, Cin, H, W = x_nchw.shape
    Cout = conv_w.shape[0]
    H2, W2 = 2 * H, 2 * W
    HW = H * W
    M = N * HW

    B1 = 2 if N % 2 == 0 else 1

    x_t = jnp.transpose(x_nchw.astype(jnp.float32), (0, 2, 3, 1))
    x3 = x_t.reshape(N, HW, Cin)
    wm = conv_w.reshape(Cout, Cin).astype(jnp.float32)

    z, s_part, sq_part = pl.pallas_call(
        _conv_stats_kernel,
        out_shape=(
            jax.ShapeDtypeStruct((N, HW, Cout), jnp.bfloat16),
            jax.ShapeDtypeStruct((N, 1, Cout), jnp.float32),
            jax.ShapeDtypeStruct((N, 1, Cout), jnp.float32),
        ),
        grid=(N // B1,),
        in_specs=[
            pl.BlockSpec((B1, HW, Cin), lambda n: (n, 0, 0)),
            pl.BlockSpec((Cout, Cin), lambda n: (0, 0)),
        ],
        out_specs=[
            pl.BlockSpec((B1, HW, Cout), lambda n: (n, 0, 0)),
            pl.BlockSpec((B1, 1, Cout), lambda n: (n, 0, 0)),
            pl.BlockSpec((B1, 1, Cout), lambda n: (n, 0, 0)),
        ],
        compiler_params=pltpu.CompilerParams(
            dimension_semantics=("parallel",)),
    )(x3, wm)

    s = jnp.sum(s_part, axis=(0, 1))
    sq = jnp.sum(sq_part, axis=(0, 1))
    mean = s / M
    var = jnp.maximum(sq / M - mean * mean, 0.0)
    scale = gamma.astype(jnp.float32) / jnp.sqrt(var + eps)
    shift = beta.astype(jnp.float32) - mean * scale

    awT = jnp.asarray(_interp_matrix_t(W, W2)).astype(jnp.bfloat16)
    h_taps = _interp_taps(H, H2)

    out_t = pl.pallas_call(
        _make_upsample_kernel(H, W, H2, W2, h_taps),
        out_shape=jax.ShapeDtypeStruct((N, H2, W2, Cout), jnp.float32),
        grid=(N // B1,),
        in_specs=[
            pl.BlockSpec((B1, HW, Cout), lambda n: (n, 0, 0)),
            pl.BlockSpec((1, Cout), lambda n: (0, 0)),
            pl.BlockSpec((1, Cout), lambda n: (0, 0)),
            pl.BlockSpec((W, W2), lambda n: (0, 0)),
        ],
        out_specs=pl.BlockSpec((B1, H2, W2, Cout), lambda n: (n, 0, 0, 0)),
        compiler_params=pltpu.CompilerParams(
            dimension_semantics=("parallel",)),
    )(z, scale.reshape(1, Cout).astype(jnp.bfloat16),
      shift.reshape(1, Cout).astype(jnp.bfloat16), awT)

    return jnp.transpose(out_t, (0, 3, 1, 2))


# speedup vs baseline: 56.9443x; 1.0610x over previous
import numpy as np
import jax
import jax.numpy as jnp
from jax.experimental import pallas as pl
from jax.experimental.pallas import tpu as pltpu


def _interp_taps(n_in, n_out):
    taps = []
    for i in range(n_out):
        if n_in == 1:
            taps.append((0, 1.0, 0.0))
            continue
        src = i * (n_in - 1) / (n_out - 1)
        lo = min(int(np.floor(src)), n_in - 2)
        f = src - lo
        taps.append((lo, 1.0 - f, f))
    return taps


def _interp_matrix_t(n_in, n_out):
    a = np.zeros((n_out, n_in), dtype=np.float32)
    for i, (lo, wl, wh) in enumerate(_interp_taps(n_in, n_out)):
        a[i, lo] += wl
        if wh:
            a[i, lo + 1] += wh
    return np.ascontiguousarray(a.T)


def _conv_stats_kernel(x_ref, w_ref, z_ref, s_ref, sq_ref):
    for i in range(x_ref.shape[0]):
        y = jax.lax.dot_general(
            x_ref[i], w_ref[...],
            dimension_numbers=(((1,), (1,)), ((), ())),
            preferred_element_type=jnp.float32)
        y = jnp.maximum(y, 0.0)
        z_ref[i] = y.astype(jnp.bfloat16)
        s_ref[i] = jnp.sum(y, axis=0, keepdims=True)
        sq_ref[i] = jnp.sum(y * y, axis=0, keepdims=True)


def _make_upsample_kernel(H, W, H2, W2, h_taps):
    def _kernel(z_ref, sc_ref, sh_ref, awT_ref, o_ref):
        c = z_ref.shape[2]
        for i in range(z_ref.shape[0]):
            t = z_ref[i] * sc_ref[...] + sh_ref[...]
            t3 = t.reshape(H, W, c)
            rows = []
            for lo, wl, wh in h_taps:
                r = t3[lo:lo + 1] * jnp.bfloat16(wl)
                if wh:
                    r = r + t3[lo + 1:lo + 2] * jnp.bfloat16(wh)
                rows.append(r)
            v = jnp.concatenate(rows, axis=0)
            vt = jnp.transpose(v, (0, 2, 1))
            u = jnp.dot(vt.reshape(H2 * c, W), awT_ref[...],
                        preferred_element_type=jnp.float32)
            u = u.astype(jnp.bfloat16)
            o = jnp.transpose(u.reshape(H2, c, W2), (0, 2, 1))
            o_ref[i] = o.astype(jnp.float32)
    return _kernel


def kernel(x_nchw, conv_w, gamma, beta, eps=1e-5):
    N, Cin, H, W = x_nchw.shape
    Cout = conv_w.shape[0]
    H2, W2 = 2 * H, 2 * W
    HW = H * W
    M = N * HW

    B1 = 4 if N % 4 == 0 else (2 if N % 2 == 0 else 1)

    x_t = jnp.transpose(x_nchw.astype(jnp.float32), (0, 2, 3, 1))
    x3 = x_t.reshape(N, HW, Cin)
    wm = conv_w.reshape(Cout, Cin).astype(jnp.float32)

    z, s_part, sq_part = pl.pallas_call(
        _conv_stats_kernel,
        out_shape=(
            jax.ShapeDtypeStruct((N, HW, Cout), jnp.bfloat16),
            jax.ShapeDtypeStruct((N, 1, Cout), jnp.float32),
            jax.ShapeDtypeStruct((N, 1, Cout), jnp.float32),
        ),
        grid=(N // B1,),
        in_specs=[
            pl.BlockSpec((B1, HW, Cin), lambda n: (n, 0, 0)),
            pl.BlockSpec((Cout, Cin), lambda n: (0, 0)),
        ],
        out_specs=[
            pl.BlockSpec((B1, HW, Cout), lambda n: (n, 0, 0)),
            pl.BlockSpec((B1, 1, Cout), lambda n: (n, 0, 0)),
            pl.BlockSpec((B1, 1, Cout), lambda n: (n, 0, 0)),
        ],
        compiler_params=pltpu.CompilerParams(
            dimension_semantics=("parallel",)),
    )(x3, wm)

    s = jnp.sum(s_part, axis=(0, 1))
    sq = jnp.sum(sq_part, axis=(0, 1))
    mean = s / M
    var = jnp.maximum(sq / M - mean * mean, 0.0)
    scale = gamma.astype(jnp.float32) / jnp.sqrt(var + eps)
    shift = beta.astype(jnp.float32) - mean * scale

    awT = jnp.asarray(_interp_matrix_t(W, W2)).astype(jnp.bfloat16)
    h_taps = _interp_taps(H, H2)

    out_t = pl.pallas_call(
        _make_upsample_kernel(H, W, H2, W2, h_taps),
        out_shape=jax.ShapeDtypeStruct((N, H2, W2, Cout), jnp.float32),
        grid=(N // B1,),
        in_specs=[
            pl.BlockSpec((B1, HW, Cout), lambda n: (n, 0, 0)),
            pl.BlockSpec((1, Cout), lambda n: (0, 0)),
            pl.BlockSpec((1, Cout), lambda n: (0, 0)),
            pl.BlockSpec((W, W2), lambda n: (0, 0)),
        ],
        out_specs=pl.BlockSpec((B1, H2, W2, Cout), lambda n: (n, 0, 0, 0)),
        compiler_params=pltpu.CompilerParams(
            dimension_semantics=("parallel",)),
    )(z, scale.reshape(1, Cout).astype(jnp.bfloat16),
      shift.reshape(1, Cout).astype(jnp.bfloat16), awT)

    return jnp.transpose(out_t, (0, 3, 1, 2))
